# baseline (speedup 1.0000x reference)
import functools

import jax
import jax.numpy as jnp
from jax import lax
from jax.experimental import pallas as pl
from jax.experimental.pallas import tpu as pltpu
from jax.experimental.pallas import tpu_sc as plsc

N = 10000
E = 160000
DIM = 32
EDGE_DIM = 16
LANES = 128
CNT = DIM

NC = 2
NS = 16
NW = NC * NS
CHUNK = 128
CPW = 40
GRP = 5
E_PAD = NW * CPW * CHUNK
NP = 10112
ROWS_PT = NP // NS
_K512 = DIM * EDGE_DIM


@functools.cache
def _mesh():
    return plsc.VectorSubcoreMesh(
        core_axis_name="c", subcore_axis_name="s",
        num_cores=NC, num_subcores=NS)


_BLK_S = 3200


def _bn_stats_body(ea_ref, sum_ref, sumsq_ref, eat_ref):
    i = pl.program_id(0)

    @pl.when(i == 0)
    def _init():
        sum_ref[...] = jnp.zeros_like(sum_ref)
        sumsq_ref[...] = jnp.zeros_like(sumsq_ref)

    ea = ea_ref[...]
    sum_ref[0:1, :] += jnp.sum(ea, axis=0, keepdims=True)
    sumsq_ref[0:1, :] += jnp.sum(ea * ea, axis=0, keepdims=True)
    eat_ref[...] = ea.T


def _bn_stats(edge_attr):
    return pl.pallas_call(
        _bn_stats_body,
        grid=(E // _BLK_S,),
        in_specs=[pl.BlockSpec((_BLK_S, EDGE_DIM), lambda i: (i, 0))],
        out_specs=(pl.BlockSpec((8, EDGE_DIM), lambda i: (0, 0)),
                   pl.BlockSpec((8, EDGE_DIM), lambda i: (0, 0)),
                   pl.BlockSpec((EDGE_DIM, _BLK_S), lambda i: (0, i))),
        out_shape=(jax.ShapeDtypeStruct((8, EDGE_DIM), jnp.float32),
                   jax.ShapeDtypeStruct((8, EDGE_DIM), jnp.float32),
                   jax.ShapeDtypeStruct((EDGE_DIM, E_PAD), jnp.float32)),
    )(edge_attr)


GCH0 = 60
GCH1 = 20
CH0_TOT = NS * GCH0
GGRP = 10


@functools.cache
def _sc_gather_call():
    return pl.kernel(
        _sc_gather_body,
        out_type=jax.ShapeDtypeStruct((E_PAD, DIM), jnp.float32),
        mesh=_mesh(),
        scratch_types=[
            pltpu.VMEM((GCH0, CHUNK), jnp.int32),
            pltpu.VMEM((GGRP * CHUNK, DIM), jnp.float32),
            pltpu.SemaphoreType.DMA,
        ],
        compiler_params=pltpu.CompilerParams(use_tc_tiling_on_sc=False),
    )


def _sc_gather(*args):
    return _sc_gather_call()(*args)


def _sc_gather_body(x_hbm, src_hbm, xj_hbm, src_v, rows_v, sem):
    c = lax.axis_index("c")
    s = lax.axis_index("s")
    cbase = jnp.where(c == 0, s * GCH0, CH0_TOT + s * GCH1)
    ngrp = jnp.where(c == 0, GCH0 // GGRP, GCH1 // GGRP)

    pltpu.sync_copy(src_hbm.at[pl.ds(pl.multiple_of(cbase, 4), GCH0)], src_v)

    def _group(g, carry):
        cps = [
            pltpu.async_copy(x_hbm.at[src_v.at[g * GGRP + j]],
                             rows_v.at[pl.ds(j * CHUNK, CHUNK)], sem)
            for j in range(GGRP)
        ]
        for cp in cps:
            cp.wait()
        off = pl.multiple_of((cbase + g * GGRP) * CHUNK, GGRP * CHUNK)
        pltpu.sync_copy(rows_v, xj_hbm.at[pl.ds(off, GGRP * CHUNK)])
        return carry

    lax.fori_loop(0, ngrp, _group, 0)


OUT_L = 64
SGRP = 10


@functools.cache
def _sc_scatter_call():
    return pl.kernel(
        _sc_scatter_body,
        out_type=jax.ShapeDtypeStruct((NC, NP, OUT_L), jnp.bfloat16),
        mesh=_mesh(),
        scratch_types=[
            pltpu.VMEM((CPW, CHUNK), jnp.int32),
            pltpu.VMEM((SGRP * CHUNK, LANES), jnp.bfloat16),
            pltpu.VMEM_SHARED((NP, LANES), jnp.bfloat16),
            pltpu.SemaphoreType.DMA,
        ],
        compiler_params=pltpu.CompilerParams(use_tc_tiling_on_sc=False),
    )


def _sc_scatter(*args):
    return _sc_scatter_call()(*args)


def _sc_scatter_body(msg_hbm, dst_hbm, zeros_hbm, acc_hbm,
                     dst_v, rows_v, acc_sh, sem):
    c = lax.axis_index("c")
    s = lax.axis_index("s")
    wid = s * NC + c
    ebase = wid * (CPW * CHUNK)

    pltpu.sync_copy(dst_hbm.at[pl.ds(wid * CPW, CPW)], dst_v)
    pltpu.sync_copy(zeros_hbm, acc_sh.at[pl.ds(s * ROWS_PT, ROWS_PT)])
    plsc.subcore_barrier()

    for g in range(CPW // SGRP):
        pltpu.sync_copy(
            msg_hbm.at[pl.ds(ebase + g * SGRP * CHUNK, SGRP * CHUNK)], rows_v)
        cps = [
            pltpu.async_copy(rows_v.at[pl.ds(j * CHUNK, CHUNK)],
                             acc_sh.at[dst_v.at[g * SGRP + j]], sem, add=True)
            for j in range(SGRP)
        ]
        for cp in cps:
            cp.wait()

    plsc.subcore_barrier()
    pltpu.sync_copy(
        acc_sh.at[pl.ds(s * ROWS_PT, ROWS_PT), pl.ds(0, OUT_L)],
        acc_hbm.at[c].at[pl.ds(s * ROWS_PT, ROWS_PT)])


_BLK_E = 2048


def _edge_msg_body(eat_ref, xjw_ref, sum_ref, sumsq_ref, gamma_ref, beta_ref,
                   r_ref, wcat_ref, s_ref, b_ref, c_ref, out_ref):
    mean = sum_ref[0:1, :] * (1.0 / E)
    var = sumsq_ref[0:1, :] * (1.0 / E) - mean * mean
    scale = gamma_ref[...] * lax.rsqrt(var + 1e-5)
    bf16 = jnp.bfloat16
    ea = ((eat_ref[...].T - mean) * scale + beta_ref[...]).astype(bf16)
    xj = xjw_ref[...].astype(bf16)
    ea_rep = jnp.dot(ea, r_ref[...], preferred_element_type=jnp.float32)
    t = jnp.dot(xj, wcat_ref[...], preferred_element_type=jnp.float32)
    u = (ea_rep * t).astype(bf16)
    out_ref[...] = (
        jnp.dot(u, s_ref[...], preferred_element_type=jnp.float32)
        + jnp.dot(xj, b_ref[...], preferred_element_type=jnp.float32)
        + c_ref[...]).astype(bf16)


def _edge_msg(ea_t, xjw, sums, sumsq, gamma, beta, r_m, wcat, s_m, b_m, c_m):
    return pl.pallas_call(
        _edge_msg_body,
        grid=(E_PAD // _BLK_E,),
        in_specs=[
            pl.BlockSpec((EDGE_DIM, _BLK_E), lambda i: (0, i)),
            pl.BlockSpec((_BLK_E, DIM), lambda i: (i, 0)),
            pl.BlockSpec((8, EDGE_DIM), lambda i: (0, 0)),
            pl.BlockSpec((8, EDGE_DIM), lambda i: (0, 0)),
            pl.BlockSpec((1, EDGE_DIM), lambda i: (0, 0)),
            pl.BlockSpec((1, EDGE_DIM), lambda i: (0, 0)),
            pl.BlockSpec((EDGE_DIM, _K512), lambda i: (0, 0)),
            pl.BlockSpec((DIM, _K512), lambda i: (0, 0)),
            pl.BlockSpec((_K512, LANES), lambda i: (0, 0)),
            pl.BlockSpec((DIM, LANES), lambda i: (0, 0)),
            pl.BlockSpec((1, LANES), lambda i: (0, 0)),
        ],
        out_specs=pl.BlockSpec((_BLK_E, LANES), lambda i: (i, 0)),
        out_shape=jax.ShapeDtypeStruct((E_PAD, LANES), jnp.bfloat16),
    )(ea_t, xjw, sums, sumsq, gamma, beta, r_m, wcat, s_m, b_m, c_m)


def _finish_body(x_ref, acc0_ref, acc1_ref, cb_ref,
                 wihT_ref, whhT_ref, bih_ref, bhh_ref, out_ref):
    x = x_ref[...]
    summed = acc0_ref[0, :, 0:DIM].astype(jnp.float32) \
        + acc1_ref[0, :, 0:DIM].astype(jnp.float32)
    cnt = acc0_ref[0, :, CNT:CNT + 1].astype(jnp.float32) \
        + acc1_ref[0, :, CNT:CNT + 1].astype(jnp.float32)
    agg = summed / jnp.maximum(cnt, 1.0)
    m = jnp.maximum(agg + cb_ref[...], 0.0)
    gi = jnp.dot(m, wihT_ref[...], preferred_element_type=jnp.float32) \
        + bih_ref[...]
    gh = jnp.dot(x, whhT_ref[...], preferred_element_type=jnp.float32) \
        + bhh_ref[...]
    r = jax.nn.sigmoid(gi[:, 0:DIM] + gh[:, 0:DIM])
    z = jax.nn.sigmoid(gi[:, DIM:2 * DIM] + gh[:, DIM:2 * DIM])
    n = jnp.tanh(gi[:, 2 * DIM:] + r * gh[:, 2 * DIM:])
    out_ref[...] = (1.0 - z) * n + z * x


def _finish(x, acc, cb, wihT, whhT, bih, bhh):
    return pl.pallas_call(
        _finish_body,
        grid=(1,),
        in_specs=[
            pl.BlockSpec((N, DIM), lambda i: (0, 0)),
            pl.BlockSpec((1, N, OUT_L), lambda i: (0, 0, 0)),
            pl.BlockSpec((1, N, OUT_L), lambda i: (1, 0, 0)),
            pl.BlockSpec((1, DIM), lambda i: (0, 0)),
            pl.BlockSpec((DIM, 3 * DIM), lambda i: (0, 0)),
            pl.BlockSpec((DIM, 3 * DIM), lambda i: (0, 0)),
            pl.BlockSpec((1, 3 * DIM), lambda i: (0, 0)),
            pl.BlockSpec((1, 3 * DIM), lambda i: (0, 0)),
        ],
        out_specs=pl.BlockSpec((N, DIM), lambda i: (0, 0)),
        out_shape=jax.ShapeDtypeStruct((N, DIM), jnp.float32),
    )(x, acc, acc, cb, wihT, whhT, bih, bhh)


def kernel(x, edge_index, edge_attr, bn_gamma, bn_beta, W_nn, b_nn,
           conv_bias, w_ih, w_hh, b_ih, b_hh):
    f32 = jnp.float32
    x = x.astype(f32)
    src = edge_index[0].astype(jnp.int32)
    dst = edge_index[1].astype(jnp.int32)

    src_rows = E_PAD // CHUNK + (GCH0 - GCH1)
    src2d = jnp.concatenate(
        [src, jnp.zeros((src_rows * CHUNK - E,), jnp.int32)]
    ).reshape(src_rows, CHUNK)
    src2d = src2d + (jnp.arange(src_rows, dtype=jnp.int32) % 8 * N)[:, None]
    x_rep = jnp.concatenate([x] * 8, axis=0)
    pad = E_PAD - E
    dst2d = jnp.concatenate(
        [dst, jnp.full((pad,), N, jnp.int32)]).reshape(E_PAD // CHUNK, CHUNK)

    zeros_acc = jnp.zeros((ROWS_PT, LANES), jnp.bfloat16)

    r_m = jnp.repeat(jnp.eye(EDGE_DIM, dtype=f32), DIM, axis=1)
    wcat = jnp.transpose(
        W_nn.astype(f32).reshape(EDGE_DIM, DIM, DIM),
        (1, 0, 2)).reshape(DIM, _K512)
    s_m = jnp.concatenate(
        [jnp.tile(jnp.eye(DIM, dtype=f32), (EDGE_DIM, 1)),
         jnp.zeros((_K512, LANES - DIM), f32)], axis=1)
    b_m = jnp.concatenate(
        [b_nn.astype(f32).reshape(DIM, DIM),
         jnp.zeros((DIM, LANES - DIM), f32)], axis=1)
    c_m = jnp.zeros((1, LANES), f32).at[0, CNT].set(1.0)

    sums, sumsq, ea_t = _bn_stats(edge_attr.astype(f32))
    xjw = _sc_gather(x_rep, src2d)
    bf16 = jnp.bfloat16
    msg = _edge_msg(ea_t, xjw, sums, sumsq,
                    bn_gamma.reshape(1, EDGE_DIM).astype(f32),
                    bn_beta.reshape(1, EDGE_DIM).astype(f32),
                    r_m.astype(bf16), wcat.astype(bf16),
                    s_m.astype(bf16), b_m.astype(bf16), c_m)
    acc = _sc_scatter(msg, dst2d, zeros_acc)

    h = _finish(x, acc,
                conv_bias.reshape(1, DIM).astype(f32),
                w_ih.T.astype(f32), w_hh.T.astype(f32),
                b_ih.reshape(1, 3 * DIM).astype(f32),
                b_hh.reshape(1, 3 * DIM).astype(f32))
    return h


# speedup vs baseline: 1.0955x; 1.0955x over previous
import functools

import jax
import jax.numpy as jnp
from jax import lax
from jax.experimental import pallas as pl
from jax.experimental.pallas import tpu as pltpu
from jax.experimental.pallas import tpu_sc as plsc

N = 10000
E = 160000
DIM = 32
EDGE_DIM = 16
LANES = 128
CNT = DIM

NC = 2
NS = 16
NW = NC * NS
CHUNK = 128
CPW = 40
GRP = 5
E_PAD = NW * CPW * CHUNK
NP = 10112
ROWS_PT = NP // NS
_K512 = DIM * EDGE_DIM


@functools.cache
def _mesh():
    return plsc.VectorSubcoreMesh(
        core_axis_name="c", subcore_axis_name="s",
        num_cores=NC, num_subcores=NS)


_BLK_S = 3200


def _bn_stats_body(ea_ref, sum_ref, sumsq_ref, eat_ref):
    i = pl.program_id(0)

    @pl.when(i == 0)
    def _init():
        sum_ref[...] = jnp.zeros_like(sum_ref)
        sumsq_ref[...] = jnp.zeros_like(sumsq_ref)

    ea = ea_ref[...]
    sum_ref[0:1, :] += jnp.sum(ea, axis=0, keepdims=True)
    sumsq_ref[0:1, :] += jnp.sum(ea * ea, axis=0, keepdims=True)
    eat_ref[...] = ea.T


def _bn_stats(edge_attr):
    return pl.pallas_call(
        _bn_stats_body,
        grid=(E // _BLK_S,),
        in_specs=[pl.BlockSpec((_BLK_S, EDGE_DIM), lambda i: (i, 0))],
        out_specs=(pl.BlockSpec((8, EDGE_DIM), lambda i: (0, 0)),
                   pl.BlockSpec((8, EDGE_DIM), lambda i: (0, 0)),
                   pl.BlockSpec((EDGE_DIM, _BLK_S), lambda i: (0, i))),
        out_shape=(jax.ShapeDtypeStruct((8, EDGE_DIM), jnp.float32),
                   jax.ShapeDtypeStruct((8, EDGE_DIM), jnp.float32),
                   jax.ShapeDtypeStruct((EDGE_DIM, E_PAD), jnp.float32)),
    )(edge_attr)


GCH0 = 60
GCH1 = 20
CH0_TOT = NS * GCH0
GGRP = 10


NREP = 8


@functools.cache
def _sc_gather_call():
    return pl.kernel(
        _sc_gather_body,
        out_type=(jax.ShapeDtypeStruct((E_PAD, DIM), jnp.float32),
                  jax.ShapeDtypeStruct((NREP * N, DIM), jnp.float32)),
        mesh=_mesh(),
        scratch_types=[
            pltpu.VMEM((GCH0, CHUNK), jnp.int32),
            pltpu.VMEM((GGRP * CHUNK, DIM), jnp.float32),
            pltpu.SemaphoreType.DMA,
        ],
        compiler_params=pltpu.CompilerParams(use_tc_tiling_on_sc=False),
    )


def _sc_gather(*args):
    return _sc_gather_call()(*args)


def _sc_gather_body(x_hbm, src_hbm, xj_hbm, xrep_hbm, src_v, rows_v, sem):
    c = lax.axis_index("c")
    s = lax.axis_index("s")
    cbase = jnp.where(c == 0, s * GCH0, CH0_TOT + s * GCH1)
    ngrp = jnp.where(c == 0, GCH0 // GGRP, GCH1 // GGRP)

    pltpu.sync_copy(src_hbm.at[pl.ds(pl.multiple_of(cbase, 4), GCH0)], src_v)

    rep = s % NREP
    half = s // NREP
    for p in range(5):
        roff = pl.multiple_of(half * 5000 + p * 1000, 8)
        pltpu.sync_copy(x_hbm.at[pl.ds(roff, 1000)],
                        rows_v.at[pl.ds(0, 1000)])
        pltpu.sync_copy(rows_v.at[pl.ds(0, 1000)],
                        xrep_hbm.at[pl.ds(pl.multiple_of(rep * N + roff, 8),
                                          1000)])
    plsc.subcore_barrier()

    def _group(g, carry):
        cps = [
            pltpu.async_copy(xrep_hbm.at[src_v.at[g * GGRP + j]],
                             rows_v.at[pl.ds(j * CHUNK, CHUNK)], sem)
            for j in range(GGRP)
        ]
        for cp in cps:
            cp.wait()
        off = pl.multiple_of((cbase + g * GGRP) * CHUNK, GGRP * CHUNK)
        pltpu.sync_copy(rows_v, xj_hbm.at[pl.ds(off, GGRP * CHUNK)])
        return carry

    lax.fori_loop(0, ngrp, _group, 0)


OUT_L = 64
SGRP = 10


@functools.cache
def _sc_scatter_call():
    return pl.kernel(
        _sc_scatter_body,
        out_type=jax.ShapeDtypeStruct((NC, NP, OUT_L), jnp.bfloat16),
        mesh=_mesh(),
        scratch_types=[
            pltpu.VMEM((CPW, CHUNK), jnp.int32),
            pltpu.VMEM((SGRP * CHUNK, LANES), jnp.bfloat16),
            pltpu.VMEM_SHARED((NP, LANES), jnp.bfloat16),
            pltpu.SemaphoreType.DMA,
        ],
        compiler_params=pltpu.CompilerParams(use_tc_tiling_on_sc=False),
    )


def _sc_scatter(*args):
    return _sc_scatter_call()(*args)


def _sc_scatter_body(msg_hbm, dst_hbm, zeros_hbm, acc_hbm,
                     dst_v, rows_v, acc_sh, sem):
    c = lax.axis_index("c")
    s = lax.axis_index("s")
    wid = s * NC + c
    ebase = wid * (CPW * CHUNK)

    pltpu.sync_copy(dst_hbm.at[pl.ds(wid * CPW, CPW)], dst_v)
    pltpu.sync_copy(zeros_hbm, acc_sh.at[pl.ds(s * ROWS_PT, ROWS_PT)])
    plsc.subcore_barrier()

    for g in range(CPW // SGRP):
        pltpu.sync_copy(
            msg_hbm.at[pl.ds(ebase + g * SGRP * CHUNK, SGRP * CHUNK)], rows_v)
        cps = [
            pltpu.async_copy(rows_v.at[pl.ds(j * CHUNK, CHUNK)],
                             acc_sh.at[dst_v.at[g * SGRP + j]], sem, add=True)
            for j in range(SGRP)
        ]
        for cp in cps:
            cp.wait()

    plsc.subcore_barrier()
    pltpu.sync_copy(
        acc_sh.at[pl.ds(s * ROWS_PT, ROWS_PT), pl.ds(0, OUT_L)],
        acc_hbm.at[c].at[pl.ds(s * ROWS_PT, ROWS_PT)])


_BLK_E = 2048


def _edge_msg_body(eat_ref, xjw_ref, sum_ref, sumsq_ref, gamma_ref, beta_ref,
                   r_ref, wcat_ref, s_ref, b_ref, c_ref, out_ref):
    mean = sum_ref[0:1, :] * (1.0 / E)
    var = sumsq_ref[0:1, :] * (1.0 / E) - mean * mean
    scale = gamma_ref[...] * lax.rsqrt(var + 1e-5)
    bf16 = jnp.bfloat16
    ea = ((eat_ref[...].T - mean) * scale + beta_ref[...]).astype(bf16)
    xj = xjw_ref[...].astype(bf16)
    ea_rep = jnp.dot(ea, r_ref[...], preferred_element_type=jnp.float32)
    t = jnp.dot(xj, wcat_ref[...], preferred_element_type=jnp.float32)
    u = (ea_rep * t).astype(bf16)
    out_ref[...] = (
        jnp.dot(u, s_ref[...], preferred_element_type=jnp.float32)
        + jnp.dot(xj, b_ref[...], preferred_element_type=jnp.float32)
        + c_ref[...]).astype(bf16)


def _edge_msg(ea_t, xjw, sums, sumsq, gamma, beta, r_m, wcat, s_m, b_m, c_m):
    return pl.pallas_call(
        _edge_msg_body,
        grid=(E_PAD // _BLK_E,),
        in_specs=[
            pl.BlockSpec((EDGE_DIM, _BLK_E), lambda i: (0, i)),
            pl.BlockSpec((_BLK_E, DIM), lambda i: (i, 0)),
            pl.BlockSpec((8, EDGE_DIM), lambda i: (0, 0)),
            pl.BlockSpec((8, EDGE_DIM), lambda i: (0, 0)),
            pl.BlockSpec((1, EDGE_DIM), lambda i: (0, 0)),
            pl.BlockSpec((1, EDGE_DIM), lambda i: (0, 0)),
            pl.BlockSpec((EDGE_DIM, _K512), lambda i: (0, 0)),
            pl.BlockSpec((DIM, _K512), lambda i: (0, 0)),
            pl.BlockSpec((_K512, LANES), lambda i: (0, 0)),
            pl.BlockSpec((DIM, LANES), lambda i: (0, 0)),
            pl.BlockSpec((1, LANES), lambda i: (0, 0)),
        ],
        out_specs=pl.BlockSpec((_BLK_E, LANES), lambda i: (i, 0)),
        out_shape=jax.ShapeDtypeStruct((E_PAD, LANES), jnp.bfloat16),
    )(ea_t, xjw, sums, sumsq, gamma, beta, r_m, wcat, s_m, b_m, c_m)


def _finish_body(x_ref, acc0_ref, acc1_ref, cb_ref,
                 wihT_ref, whhT_ref, bih_ref, bhh_ref, out_ref):
    x = x_ref[...]
    summed = acc0_ref[0, :, 0:DIM].astype(jnp.float32) \
        + acc1_ref[0, :, 0:DIM].astype(jnp.float32)
    cnt = acc0_ref[0, :, CNT:CNT + 1].astype(jnp.float32) \
        + acc1_ref[0, :, CNT:CNT + 1].astype(jnp.float32)
    agg = summed / jnp.maximum(cnt, 1.0)
    m = jnp.maximum(agg + cb_ref[...], 0.0)
    gi = jnp.dot(m, wihT_ref[...], preferred_element_type=jnp.float32) \
        + bih_ref[...]
    gh = jnp.dot(x, whhT_ref[...], preferred_element_type=jnp.float32) \
        + bhh_ref[...]
    r = jax.nn.sigmoid(gi[:, 0:DIM] + gh[:, 0:DIM])
    z = jax.nn.sigmoid(gi[:, DIM:2 * DIM] + gh[:, DIM:2 * DIM])
    n = jnp.tanh(gi[:, 2 * DIM:] + r * gh[:, 2 * DIM:])
    out_ref[...] = (1.0 - z) * n + z * x


def _finish(x, acc, cb, wihT, whhT, bih, bhh):
    return pl.pallas_call(
        _finish_body,
        grid=(1,),
        in_specs=[
            pl.BlockSpec((N, DIM), lambda i: (0, 0)),
            pl.BlockSpec((1, N, OUT_L), lambda i: (0, 0, 0)),
            pl.BlockSpec((1, N, OUT_L), lambda i: (1, 0, 0)),
            pl.BlockSpec((1, DIM), lambda i: (0, 0)),
            pl.BlockSpec((DIM, 3 * DIM), lambda i: (0, 0)),
            pl.BlockSpec((DIM, 3 * DIM), lambda i: (0, 0)),
            pl.BlockSpec((1, 3 * DIM), lambda i: (0, 0)),
            pl.BlockSpec((1, 3 * DIM), lambda i: (0, 0)),
        ],
        out_specs=pl.BlockSpec((N, DIM), lambda i: (0, 0)),
        out_shape=jax.ShapeDtypeStruct((N, DIM), jnp.float32),
    )(x, acc, acc, cb, wihT, whhT, bih, bhh)


def kernel(x, edge_index, edge_attr, bn_gamma, bn_beta, W_nn, b_nn,
           conv_bias, w_ih, w_hh, b_ih, b_hh):
    f32 = jnp.float32
    x = x.astype(f32)
    src = edge_index[0].astype(jnp.int32)
    dst = edge_index[1].astype(jnp.int32)

    src_rows = E_PAD // CHUNK + (GCH0 - GCH1)
    src2d = jnp.concatenate(
        [src, jnp.zeros((src_rows * CHUNK - E,), jnp.int32)]
    ).reshape(src_rows, CHUNK)
    src2d = src2d + (jnp.arange(src_rows, dtype=jnp.int32) % NREP * N)[:, None]
    pad = E_PAD - E
    dst2d = jnp.concatenate(
        [dst, jnp.full((pad,), N, jnp.int32)]).reshape(E_PAD // CHUNK, CHUNK)

    zeros_acc = jnp.zeros((ROWS_PT, LANES), jnp.bfloat16)

    r_m = jnp.repeat(jnp.eye(EDGE_DIM, dtype=f32), DIM, axis=1)
    wcat = jnp.transpose(
        W_nn.astype(f32).reshape(EDGE_DIM, DIM, DIM),
        (1, 0, 2)).reshape(DIM, _K512)
    s_m = jnp.concatenate(
        [jnp.tile(jnp.eye(DIM, dtype=f32), (EDGE_DIM, 1)),
         jnp.zeros((_K512, LANES - DIM), f32)], axis=1)
    b_m = jnp.concatenate(
        [b_nn.astype(f32).reshape(DIM, DIM),
         jnp.zeros((DIM, LANES - DIM), f32)], axis=1)
    c_m = jnp.zeros((1, LANES), f32).at[0, CNT].set(1.0)

    sums, sumsq, ea_t = _bn_stats(edge_attr.astype(f32))
    xjw, _ = _sc_gather(x, src2d)
    bf16 = jnp.bfloat16
    msg = _edge_msg(ea_t, xjw, sums, sumsq,
                    bn_gamma.reshape(1, EDGE_DIM).astype(f32),
                    bn_beta.reshape(1, EDGE_DIM).astype(f32),
                    r_m.astype(bf16), wcat.astype(bf16),
                    s_m.astype(bf16), b_m.astype(bf16), c_m)
    acc = _sc_scatter(msg, dst2d, zeros_acc)

    h = _finish(x, acc,
                conv_bias.reshape(1, DIM).astype(f32),
                w_ih.T.astype(f32), w_hh.T.astype(f32),
                b_ih.reshape(1, 3 * DIM).astype(f32),
                b_hh.reshape(1, 3 * DIM).astype(f32))
    return h


# speedup vs baseline: 1.1464x; 1.0464x over previous
import functools

import jax
import jax.numpy as jnp
from jax import lax
from jax.experimental import pallas as pl
from jax.experimental.pallas import tpu as pltpu
from jax.experimental.pallas import tpu_sc as plsc

N = 10000
E = 160000
DIM = 32
EDGE_DIM = 16
LANES = 128
CNT = DIM

NC = 2
NS = 16
NW = NC * NS
CHUNK = 128
CPW = 40
GRP = 5
E_PAD = NW * CPW * CHUNK
NP = 10112
ROWS_PT = NP // NS
_K512 = DIM * EDGE_DIM


@functools.cache
def _mesh():
    return plsc.VectorSubcoreMesh(
        core_axis_name="c", subcore_axis_name="s",
        num_cores=NC, num_subcores=NS)


_BLK_S = 3200


def _bn_stats_body(ea_ref, sum_ref, sumsq_ref, eat_ref):
    i = pl.program_id(0)

    @pl.when(i == 0)
    def _init():
        sum_ref[...] = jnp.zeros_like(sum_ref)
        sumsq_ref[...] = jnp.zeros_like(sumsq_ref)

    ea = ea_ref[...]
    sum_ref[0:1, :] += jnp.sum(ea, axis=0, keepdims=True)
    sumsq_ref[0:1, :] += jnp.sum(ea * ea, axis=0, keepdims=True)
    eat_ref[...] = ea.T


def _bn_stats(edge_attr):
    return pl.pallas_call(
        _bn_stats_body,
        grid=(E // _BLK_S,),
        in_specs=[pl.BlockSpec((_BLK_S, EDGE_DIM), lambda i: (i, 0))],
        out_specs=(pl.BlockSpec((8, EDGE_DIM), lambda i: (0, 0)),
                   pl.BlockSpec((8, EDGE_DIM), lambda i: (0, 0)),
                   pl.BlockSpec((EDGE_DIM, _BLK_S), lambda i: (0, i))),
        out_shape=(jax.ShapeDtypeStruct((8, EDGE_DIM), jnp.float32),
                   jax.ShapeDtypeStruct((8, EDGE_DIM), jnp.float32),
                   jax.ShapeDtypeStruct((EDGE_DIM, E_PAD), jnp.float32)),
    )(edge_attr)


GCH0 = 60
GCH1 = 20
CH0_TOT = NS * GCH0
GGRP = 10


NREP = 8


@functools.cache
def _sc_gather_call():
    return pl.kernel(
        _sc_gather_body,
        out_type=(jax.ShapeDtypeStruct((E_PAD, DIM), jnp.float32),
                  jax.ShapeDtypeStruct((NREP * N, DIM), jnp.float32)),
        mesh=_mesh(),
        scratch_types=[
            pltpu.VMEM((GCH0, CHUNK), jnp.int32),
            pltpu.VMEM((GGRP * CHUNK, DIM), jnp.float32),
            pltpu.SemaphoreType.DMA,
        ],
        compiler_params=pltpu.CompilerParams(use_tc_tiling_on_sc=False),
    )


def _sc_gather(*args):
    return _sc_gather_call()(*args)


def _sc_gather_body(x_hbm, src_hbm, xj_hbm, xrep_hbm, src_v, rows_v, sem):
    c = lax.axis_index("c")
    s = lax.axis_index("s")
    cbase = jnp.where(c == 0, s * GCH0, CH0_TOT + s * GCH1)
    ngrp = jnp.where(c == 0, GCH0 // GGRP, GCH1 // GGRP)

    pltpu.sync_copy(src_hbm.at[pl.ds(pl.multiple_of(cbase, 4), GCH0)], src_v)

    rep = s % NREP
    half = s // NREP
    for p in range(5):
        roff = pl.multiple_of(half * 5000 + p * 1000, 8)
        pltpu.sync_copy(x_hbm.at[pl.ds(roff, 1000)],
                        rows_v.at[pl.ds(0, 1000)])
        pltpu.sync_copy(rows_v.at[pl.ds(0, 1000)],
                        xrep_hbm.at[pl.ds(pl.multiple_of(rep * N + roff, 8),
                                          1000)])
    plsc.subcore_barrier()

    def _group(g, carry):
        cps = [
            pltpu.async_copy(xrep_hbm.at[src_v.at[g * GGRP + j]],
                             rows_v.at[pl.ds(j * CHUNK, CHUNK)], sem)
            for j in range(GGRP)
        ]
        for cp in cps:
            cp.wait()
        off = pl.multiple_of((cbase + g * GGRP) * CHUNK, GGRP * CHUNK)
        pltpu.sync_copy(rows_v, xj_hbm.at[pl.ds(off, GGRP * CHUNK)])
        return carry

    lax.fori_loop(0, ngrp, _group, 0)


NPH = NP // NC
NPH_A = 5120
ROWS_SC = NPH_A // NS
CPT = E_PAD // CHUNK // NS
OUT_L = 48
SGRP = 2


@functools.cache
def _sc_scatter_call():
    return pl.kernel(
        _sc_scatter_body,
        out_type=jax.ShapeDtypeStruct((NC, NPH_A, OUT_L), jnp.float32),
        mesh=_mesh(),
        scratch_types=[
            pltpu.VMEM((CPT, CHUNK), jnp.int32),
            pltpu.VMEM((SGRP * CHUNK, LANES), jnp.float32),
            pltpu.VMEM_SHARED((NPH_A, LANES), jnp.float32),
            pltpu.SemaphoreType.DMA,
        ],
        compiler_params=pltpu.CompilerParams(use_tc_tiling_on_sc=False),
    )


def _sc_scatter(*args):
    return _sc_scatter_call()(*args)


def _sc_scatter_body(msg_hbm, dst_hbm, acc_hbm,
                     dst_v, rows_v, acc_sh, sem):
    c = lax.axis_index("c")
    s = lax.axis_index("s")
    base = c * NPH
    ebase = s * (CPT * CHUNK)

    pltpu.sync_copy(dst_hbm.at[pl.ds(s * CPT, CPT)], dst_v)

    def _zero(i, carry):
        r = i // 8
        k = pl.multiple_of((i % 8) * 16, 16)
        rows_v[r, pl.ds(k, 16)] = jnp.zeros((16,), jnp.float32)
        return carry

    lax.fori_loop(0, (ROWS_SC // 2) * 8, _zero, 0)
    pltpu.sync_copy(rows_v.at[pl.ds(0, ROWS_SC // 2)],
                    acc_sh.at[pl.ds(s * ROWS_SC, ROWS_SC // 2)])
    pltpu.sync_copy(rows_v.at[pl.ds(0, ROWS_SC // 2)],
                    acc_sh.at[pl.ds(s * ROWS_SC + ROWS_SC // 2,
                                    ROWS_SC // 2)])

    def _xform(i, carry):
        r = i // 8
        k = pl.multiple_of((i % 8) * 16, 16)
        v = dst_v[r, pl.ds(k, 16)] - base
        ok = (v >= 0) & (v < NPH)
        dst_v[r, pl.ds(k, 16)] = jnp.where(ok, v, NPH)
        return carry

    lax.fori_loop(0, CPT * 8, _xform, 0)
    plsc.subcore_barrier()

    for g in range(CPT // SGRP):
        pltpu.sync_copy(
            msg_hbm.at[pl.ds(ebase + g * SGRP * CHUNK, SGRP * CHUNK)], rows_v)
        cps = [
            pltpu.async_copy(rows_v.at[pl.ds(j * CHUNK, CHUNK)],
                             acc_sh.at[dst_v.at[g * SGRP + j]], sem, add=True)
            for j in range(SGRP)
        ]
        for cp in cps:
            cp.wait()

    plsc.subcore_barrier()
    pltpu.sync_copy(
        acc_sh.at[pl.ds(s * ROWS_SC, ROWS_SC), pl.ds(0, OUT_L)],
        acc_hbm.at[c].at[pl.ds(s * ROWS_SC, ROWS_SC)])


_BLK_E = 2048


def _edge_msg_body(eat_ref, xjw_ref, sum_ref, sumsq_ref, gamma_ref, beta_ref,
                   r_ref, wcat_ref, s_ref, b_ref, c_ref, out_ref):
    mean = sum_ref[0:1, :] * (1.0 / E)
    var = sumsq_ref[0:1, :] * (1.0 / E) - mean * mean
    scale = gamma_ref[...] * lax.rsqrt(var + 1e-5)
    bf16 = jnp.bfloat16
    ea = ((eat_ref[...].T - mean) * scale + beta_ref[...]).astype(bf16)
    xj = xjw_ref[...].astype(bf16)
    ea_rep = jnp.dot(ea, r_ref[...], preferred_element_type=jnp.float32)
    t = jnp.dot(xj, wcat_ref[...], preferred_element_type=jnp.float32)
    u = (ea_rep * t).astype(bf16)
    out_ref[...] = (
        jnp.dot(u, s_ref[...], preferred_element_type=jnp.float32)
        + jnp.dot(xj, b_ref[...], preferred_element_type=jnp.float32)
        + c_ref[...])


def _edge_msg(ea_t, xjw, sums, sumsq, gamma, beta, r_m, wcat, s_m, b_m, c_m):
    return pl.pallas_call(
        _edge_msg_body,
        grid=(E_PAD // _BLK_E,),
        in_specs=[
            pl.BlockSpec((EDGE_DIM, _BLK_E), lambda i: (0, i)),
            pl.BlockSpec((_BLK_E, DIM), lambda i: (i, 0)),
            pl.BlockSpec((8, EDGE_DIM), lambda i: (0, 0)),
            pl.BlockSpec((8, EDGE_DIM), lambda i: (0, 0)),
            pl.BlockSpec((1, EDGE_DIM), lambda i: (0, 0)),
            pl.BlockSpec((1, EDGE_DIM), lambda i: (0, 0)),
            pl.BlockSpec((EDGE_DIM, _K512), lambda i: (0, 0)),
            pl.BlockSpec((DIM, _K512), lambda i: (0, 0)),
            pl.BlockSpec((_K512, LANES), lambda i: (0, 0)),
            pl.BlockSpec((DIM, LANES), lambda i: (0, 0)),
            pl.BlockSpec((1, LANES), lambda i: (0, 0)),
        ],
        out_specs=pl.BlockSpec((_BLK_E, LANES), lambda i: (i, 0)),
        out_shape=jax.ShapeDtypeStruct((E_PAD, LANES), jnp.float32),
    )(ea_t, xjw, sums, sumsq, gamma, beta, r_m, wcat, s_m, b_m, c_m)


def _finish_body(x_ref, acc0_ref, acc1_ref, cb_ref,
                 wihT_ref, whhT_ref, bih_ref, bhh_ref, out_ref):
    x = x_ref[...]
    summed = jnp.concatenate(
        [acc0_ref[0, :, 0:DIM], acc1_ref[0, :, 0:DIM]], axis=0)
    cnt = jnp.concatenate(
        [acc0_ref[0, :, CNT:CNT + 1], acc1_ref[0, :, CNT:CNT + 1]], axis=0)
    agg = summed / jnp.maximum(cnt, 1.0)
    m = jnp.maximum(agg + cb_ref[...], 0.0)
    gi = jnp.dot(m, wihT_ref[...], preferred_element_type=jnp.float32) \
        + bih_ref[...]
    gh = jnp.dot(x, whhT_ref[...], preferred_element_type=jnp.float32) \
        + bhh_ref[...]
    r = jax.nn.sigmoid(gi[:, 0:DIM] + gh[:, 0:DIM])
    z = jax.nn.sigmoid(gi[:, DIM:2 * DIM] + gh[:, DIM:2 * DIM])
    n = jnp.tanh(gi[:, 2 * DIM:] + r * gh[:, 2 * DIM:])
    out_ref[...] = (1.0 - z) * n + z * x


def _finish(x, acc, cb, wihT, whhT, bih, bhh):
    return pl.pallas_call(
        _finish_body,
        grid=(1,),
        in_specs=[
            pl.BlockSpec((N, DIM), lambda i: (0, 0)),
            pl.BlockSpec((1, NPH, OUT_L), lambda i: (0, 0, 0)),
            pl.BlockSpec((1, N - NPH, OUT_L), lambda i: (1, 0, 0)),
            pl.BlockSpec((1, DIM), lambda i: (0, 0)),
            pl.BlockSpec((DIM, 3 * DIM), lambda i: (0, 0)),
            pl.BlockSpec((DIM, 3 * DIM), lambda i: (0, 0)),
            pl.BlockSpec((1, 3 * DIM), lambda i: (0, 0)),
            pl.BlockSpec((1, 3 * DIM), lambda i: (0, 0)),
        ],
        out_specs=pl.BlockSpec((N, DIM), lambda i: (0, 0)),
        out_shape=jax.ShapeDtypeStruct((N, DIM), jnp.float32),
    )(x, acc, acc, cb, wihT, whhT, bih, bhh)


def kernel(x, edge_index, edge_attr, bn_gamma, bn_beta, W_nn, b_nn,
           conv_bias, w_ih, w_hh, b_ih, b_hh):
    f32 = jnp.float32
    x = x.astype(f32)
    src = edge_index[0].astype(jnp.int32)
    dst = edge_index[1].astype(jnp.int32)

    src_rows = E_PAD // CHUNK + (GCH0 - GCH1)
    src2d = jnp.concatenate(
        [src, jnp.zeros((src_rows * CHUNK - E,), jnp.int32)]
    ).reshape(src_rows, CHUNK)
    src2d = src2d + (jnp.arange(src_rows, dtype=jnp.int32) % NREP * N)[:, None]
    pad = E_PAD - E
    dst2d = jnp.concatenate(
        [dst, jnp.full((8 * E_PAD // CHUNK * CHUNK - E,), N, jnp.int32)]
    ).reshape(8 * E_PAD // CHUNK, CHUNK)


    r_m = jnp.repeat(jnp.eye(EDGE_DIM, dtype=f32), DIM, axis=1)
    wcat = jnp.transpose(
        W_nn.astype(f32).reshape(EDGE_DIM, DIM, DIM),
        (1, 0, 2)).reshape(DIM, _K512)
    s_m = jnp.concatenate(
        [jnp.tile(jnp.eye(DIM, dtype=f32), (EDGE_DIM, 1)),
         jnp.zeros((_K512, LANES - DIM), f32)], axis=1)
    b_m = jnp.concatenate(
        [b_nn.astype(f32).reshape(DIM, DIM),
         jnp.zeros((DIM, LANES - DIM), f32)], axis=1)
    c_m = jnp.zeros((1, LANES), f32).at[0, CNT].set(1.0)

    sums, sumsq, ea_t = _bn_stats(edge_attr.astype(f32))
    xjw, _ = _sc_gather(x, src2d)
    bf16 = jnp.bfloat16
    msg = _edge_msg(ea_t, xjw, sums, sumsq,
                    bn_gamma.reshape(1, EDGE_DIM).astype(f32),
                    bn_beta.reshape(1, EDGE_DIM).astype(f32),
                    r_m.astype(bf16), wcat.astype(bf16),
                    s_m.astype(bf16), b_m.astype(bf16), c_m)
    acc = _sc_scatter(msg, dst2d)

    h = _finish(x, acc,
                conv_bias.reshape(1, DIM).astype(f32),
                w_ih.T.astype(f32), w_hh.T.astype(f32),
                b_ih.reshape(1, 3 * DIM).astype(f32),
                b_hh.reshape(1, 3 * DIM).astype(f32))
    return h
